# baseline (speedup 1.0000x reference)
import functools

import jax
import jax.numpy as jnp
from jax import lax
from jax.experimental import pallas as pl
from jax.experimental.pallas import tpu as pltpu
from jax.experimental.pallas import tpu_sc as plsc

NC = 2
NS = 16
EBLK = 128


def _sc_aggregate(x, src, dst):
    n, d = x.shape
    nw = NC * NS
    nblk = dst.shape[0]
    bpw = nblk // nw
    nk2 = bpw // 2
    epw = bpw * EBLK

    rows_per = (n // NS) & ~7
    tail_r0 = rows_per * NS
    tail_n = n - tail_r0

    mesh = plsc.VectorSubcoreMesh(core_axis_name="c", subcore_axis_name="s")

    @functools.partial(
        pl.kernel,
        mesh=mesh,
        out_type=jax.ShapeDtypeStruct((NC, n, d), jnp.float32),
        scratch_types=[
            pltpu.VMEM((epw,), jnp.int32),
            pltpu.VMEM((1, EBLK), jnp.int32),
            pltpu.VMEM((1, EBLK), jnp.int32),
            pltpu.VMEM((EBLK, d), jnp.float32),
            pltpu.VMEM((EBLK, d), jnp.float32),
            pltpu.VMEM_SHARED((n + 8, d), jnp.float32),
            pltpu.SemaphoreType.DMA,
            pltpu.SemaphoreType.DMA,
            pltpu.SemaphoreType.DMA,
            pltpu.SemaphoreType.DMA,
            pltpu.SemaphoreType.DMA,
        ],
    )
    def sc_kernel(x_hbm, src_hbm, dst_hbm, out_hbm, src_v, db_a, db_b, m_a,
                  m_b, agg_sh, sem_a, sem_b, sem_da, sem_db, sem_i):
        c = lax.axis_index("c")
        s = lax.axis_index("s")
        w = c * NS + s

        pltpu.make_async_copy(src_hbm.at[pl.ds(w * epw, epw)], src_v,
                              sem_i).start()

        pltpu.sync_copy(x_hbm.at[pl.ds(s * rows_per, rows_per)],
                        agg_sh.at[pl.ds(s * rows_per, rows_per)])
        if tail_n:
            @pl.when(s == 0)
            def _():
                pltpu.sync_copy(x_hbm.at[pl.ds(tail_r0, tail_n)],
                                agg_sh.at[pl.ds(tail_r0, tail_n)])
        pltpu.make_async_copy(src_hbm.at[pl.ds(w * epw, epw)], src_v,
                              sem_i).wait()
        plsc.subcore_barrier()

        blk0 = w * bpw

        def gather(b, buf, sem):
            return pltpu.make_async_copy(
                x_hbm.at[src_v.at[pl.ds(b * EBLK, EBLK)]], buf, sem)

        def dstcopy(b, buf, sem):
            return pltpu.make_async_copy(dst_hbm.at[pl.ds(blk0 + b, 1)], buf,
                                         sem)

        def scatter(buf, db):
            pltpu.sync_copy(buf, agg_sh.at[db.at[0]], add=True)

        gather(0, m_a, sem_a).start()
        dstcopy(0, db_a, sem_da).start()

        @pl.loop(0, nk2)
        def _(k2):
            b0 = 2 * k2
            b1 = b0 + 1
            gather(b1, m_b, sem_b).start()
            dstcopy(b1, db_b, sem_db).start()
            gather(b0, m_a, sem_a).wait()
            dstcopy(b0, db_a, sem_da).wait()
            scatter(m_a, db_a)

            @pl.when(k2 < nk2 - 1)
            def _():
                gather(b0 + 2, m_a, sem_a).start()
                dstcopy(b0 + 2, db_a, sem_da).start()
            gather(b1, m_b, sem_b).wait()
            dstcopy(b1, db_b, sem_db).wait()
            scatter(m_b, db_b)

        plsc.subcore_barrier()

        pltpu.sync_copy(agg_sh.at[pl.ds(s * rows_per, rows_per)],
                        out_hbm.at[c, pl.ds(s * rows_per, rows_per)])
        if tail_n:
            @pl.when(s == 0)
            def _():
                pltpu.sync_copy(agg_sh.at[pl.ds(tail_r0, tail_n)],
                                out_hbm.at[c, pl.ds(tail_r0, tail_n)])

    return sc_kernel(x, src, dst)


def _tc_body(scale_ref, x_ref, p0_ref, p1_ref, w1_ref, b1_ref, w2_ref, b2_ref,
             o_ref):
    h = x_ref[...] * scale_ref[0, 0] + p0_ref[...] + p1_ref[...]
    h = jnp.dot(h, w1_ref[...], preferred_element_type=jnp.float32,
                precision=lax.Precision.HIGHEST) + b1_ref[...]
    h = jnp.maximum(h, 0.0)
    o_ref[...] = jnp.dot(h, w2_ref[...], preferred_element_type=jnp.float32,
                         precision=lax.Precision.HIGHEST) + b2_ref[...]


def _tc_mlp(x, p0, p1, W1, b1, W2, b2, eps):
    n, d = x.shape
    blk = 1000
    grid = (n // blk,)
    scale = (eps - 1.0).reshape(1, 1)
    return pl.pallas_call(
        _tc_body,
        grid=grid,
        in_specs=[
            pl.BlockSpec((1, 1), lambda i: (0, 0)),
            pl.BlockSpec((blk, d), lambda i: (i, 0)),
            pl.BlockSpec((blk, d), lambda i: (i, 0)),
            pl.BlockSpec((blk, d), lambda i: (i, 0)),
            pl.BlockSpec((d, d), lambda i: (0, 0)),
            pl.BlockSpec((1, d), lambda i: (0, 0)),
            pl.BlockSpec((d, d), lambda i: (0, 0)),
            pl.BlockSpec((1, d), lambda i: (0, 0)),
        ],
        out_specs=pl.BlockSpec((blk, d), lambda i: (i, 0)),
        out_shape=jax.ShapeDtypeStruct((n, d), jnp.float32),
    )(scale, x, p0, p1, W1, b1.reshape(1, d), W2, b2.reshape(1, d))


def kernel(x, edge_index, W1, b1, W2, b2, eps):
    n = x.shape[0]
    e = edge_index.shape[1]
    nw = NC * NS
    unit = 2 * nw * EBLK
    e_pad = ((e + unit - 1) // unit) * unit
    src = jnp.concatenate([edge_index[0], jnp.zeros((e_pad - e,), jnp.int32)])
    dst = jnp.concatenate([edge_index[1], jnp.full((e_pad - e,), n, jnp.int32)])
    dst = dst.reshape(e_pad // EBLK, EBLK)
    partials = _sc_aggregate(x, src, dst)
    return _tc_mlp(x, partials[0], partials[1], W1, b1, W2, b2, eps)


# speedup vs baseline: 1.0491x; 1.0491x over previous
import functools

import jax
import jax.numpy as jnp
from jax import lax
from jax.experimental import pallas as pl
from jax.experimental.pallas import tpu as pltpu
from jax.experimental.pallas import tpu_sc as plsc

NC = 2
NS = 16
EBLK = 128


def _sc_aggregate(x, src, dst, n):
    d = x.shape[1]
    nw = NC * NS
    nblk = dst.shape[0]
    bpw = nblk // nw
    nk2 = bpw // 2
    epw = bpw * EBLK

    rows_per = (n // NS) & ~7
    tail_r0 = rows_per * NS
    tail_n = n - tail_r0

    mesh = plsc.VectorSubcoreMesh(core_axis_name="c", subcore_axis_name="s")

    @functools.partial(
        pl.kernel,
        mesh=mesh,
        out_type=jax.ShapeDtypeStruct((NC, n, d), jnp.float32),
        scratch_types=[
            pltpu.VMEM((epw,), jnp.int32),
            pltpu.VMEM((1, EBLK), jnp.int32),
            pltpu.VMEM((1, EBLK), jnp.int32),
            pltpu.VMEM((EBLK, d), jnp.float32),
            pltpu.VMEM((EBLK, d), jnp.float32),
            pltpu.VMEM_SHARED((n, d), jnp.float32),
            pltpu.SemaphoreType.DMA,
            pltpu.SemaphoreType.DMA,
            pltpu.SemaphoreType.DMA,
            pltpu.SemaphoreType.DMA,
            pltpu.SemaphoreType.DMA,
        ],
    )
    def sc_kernel(x_hbm, src_hbm, dst_hbm, out_hbm, src_v, db_a, db_b, m_a,
                  m_b, agg_sh, sem_a, sem_b, sem_da, sem_db, sem_i):
        c = lax.axis_index("c")
        s = lax.axis_index("s")
        w = c * NS + s

        pltpu.make_async_copy(src_hbm.at[pl.ds(w * epw, epw)], src_v,
                              sem_i).start()

        pltpu.sync_copy(x_hbm.at[pl.ds(s * rows_per, rows_per)],
                        agg_sh.at[pl.ds(s * rows_per, rows_per)])
        if tail_n:
            @pl.when(s == 0)
            def _():
                pltpu.sync_copy(x_hbm.at[pl.ds(tail_r0, tail_n)],
                                agg_sh.at[pl.ds(tail_r0, tail_n)])
        pltpu.make_async_copy(src_hbm.at[pl.ds(w * epw, epw)], src_v,
                              sem_i).wait()
        plsc.subcore_barrier()

        blk0 = w * bpw

        def gather(b, buf, sem):
            return pltpu.make_async_copy(
                x_hbm.at[src_v.at[pl.ds(b * EBLK, EBLK)]], buf, sem)

        def dstcopy(b, buf, sem):
            return pltpu.make_async_copy(dst_hbm.at[pl.ds(blk0 + b, 1)], buf,
                                         sem)

        def scatter(buf, db):
            pltpu.sync_copy(buf, agg_sh.at[db.at[0]], add=True)

        gather(0, m_a, sem_a).start()
        dstcopy(0, db_a, sem_da).start()

        @pl.loop(0, nk2)
        def _(k2):
            b0 = 2 * k2
            b1 = b0 + 1
            gather(b1, m_b, sem_b).start()
            dstcopy(b1, db_b, sem_db).start()
            gather(b0, m_a, sem_a).wait()
            dstcopy(b0, db_a, sem_da).wait()
            scatter(m_a, db_a)

            @pl.when(k2 < nk2 - 1)
            def _():
                gather(b0 + 2, m_a, sem_a).start()
                dstcopy(b0 + 2, db_a, sem_da).start()
            gather(b1, m_b, sem_b).wait()
            dstcopy(b1, db_b, sem_db).wait()
            scatter(m_b, db_b)

        plsc.subcore_barrier()

        pltpu.sync_copy(agg_sh.at[pl.ds(s * rows_per, rows_per)],
                        out_hbm.at[c, pl.ds(s * rows_per, rows_per)])
        if tail_n:
            @pl.when(s == 0)
            def _():
                pltpu.sync_copy(agg_sh.at[pl.ds(tail_r0, tail_n)],
                                out_hbm.at[c, pl.ds(tail_r0, tail_n)])

    return sc_kernel(x, src, dst)


def _tc_body(scale_ref, x_ref, p0_ref, p1_ref, w1_ref, b1_ref, w2_ref, b2_ref,
             o_ref):
    h = x_ref[...] * scale_ref[0, 0] + p0_ref[...] + p1_ref[...]
    h = jnp.dot(h, w1_ref[...], preferred_element_type=jnp.float32,
                precision=lax.Precision.HIGHEST) + b1_ref[...]
    h = jnp.maximum(h, 0.0)
    o_ref[...] = jnp.dot(h, w2_ref[...], preferred_element_type=jnp.float32,
                         precision=lax.Precision.HIGHEST) + b2_ref[...]


def _tc_mlp(x, p0, p1, W1, b1, W2, b2, eps):
    n, d = x.shape
    blk = 1000
    grid = (n // blk,)
    scale = (eps - 1.0).reshape(1, 1)
    return pl.pallas_call(
        _tc_body,
        grid=grid,
        in_specs=[
            pl.BlockSpec((1, 1), lambda i: (0, 0)),
            pl.BlockSpec((blk, d), lambda i: (i, 0)),
            pl.BlockSpec((blk, d), lambda i: (i, 0)),
            pl.BlockSpec((blk, d), lambda i: (i, 0)),
            pl.BlockSpec((d, d), lambda i: (0, 0)),
            pl.BlockSpec((1, d), lambda i: (0, 0)),
            pl.BlockSpec((d, d), lambda i: (0, 0)),
            pl.BlockSpec((1, d), lambda i: (0, 0)),
        ],
        out_specs=pl.BlockSpec((blk, d), lambda i: (i, 0)),
        out_shape=jax.ShapeDtypeStruct((n, d), jnp.float32),
    )(scale, x, p0, p1, W1, b1.reshape(1, d), W2, b2.reshape(1, d))


def kernel(x, edge_index, W1, b1, W2, b2, eps):
    n = x.shape[0]
    e = edge_index.shape[1]
    nw = NC * NS
    unit = 2 * nw * EBLK
    e_pad = ((e + unit - 1) // unit) * unit
    pad = e_pad - e
    src = jnp.concatenate([edge_index[0], jnp.full((pad,), n, jnp.int32)])
    dst = jnp.concatenate(
        [edge_index[1], (jnp.arange(pad, dtype=jnp.int32) % n)])
    dst = dst.reshape(e_pad // EBLK, EBLK)
    x_g = jnp.concatenate([x, jnp.zeros((8, x.shape[1]), jnp.float32)])
    partials = _sc_aggregate(x_g, src, dst, n)
    return _tc_mlp(x, partials[0], partials[1], W1, b1, W2, b2, eps)


# speedup vs baseline: 3.3561x; 3.1992x over previous
import functools

import jax
import jax.numpy as jnp
from jax import lax
from jax.experimental import pallas as pl
from jax.experimental.pallas import tpu as pltpu
from jax.experimental.pallas import tpu_sc as plsc

NC = 2
NS = 16
EBLK = 128


def _sc_aggregate(x, src, dst, n):
    d = x.shape[1]
    nw = NC * NS
    nblk = dst.shape[0]
    bpw = nblk // nw
    nk2 = bpw // 2
    epw = bpw * EBLK

    rows_per = (n // NS) & ~7
    tail_r0 = rows_per * NS
    tail_n = n - tail_r0

    mesh = plsc.VectorSubcoreMesh(core_axis_name="c", subcore_axis_name="s")

    @functools.partial(
        pl.kernel,
        mesh=mesh,
        out_type=jax.ShapeDtypeStruct((NC, n, d), jnp.float32),
        scratch_types=[
            pltpu.VMEM((epw,), jnp.int32),
            pltpu.VMEM((1, EBLK), jnp.int32),
            pltpu.VMEM((1, EBLK), jnp.int32),
            pltpu.VMEM((EBLK, d), jnp.float32),
            pltpu.VMEM((EBLK, d), jnp.float32),
            pltpu.VMEM_SHARED((n + EBLK, d), jnp.float32),
            pltpu.SemaphoreType.DMA,
            pltpu.SemaphoreType.DMA,
            pltpu.SemaphoreType.DMA,
            pltpu.SemaphoreType.DMA,
            pltpu.SemaphoreType.DMA,
        ],
    )
    def sc_kernel(x_hbm, src_hbm, dst_hbm, out_hbm, src_v, db_a, db_b, m_a,
                  m_b, agg_sh, sem_a, sem_b, sem_da, sem_db, sem_i):
        c = lax.axis_index("c")
        s = lax.axis_index("s")
        w = c * NS + s

        pltpu.make_async_copy(src_hbm.at[pl.ds(w * epw, epw)], src_v,
                              sem_i).start()

        pltpu.sync_copy(x_hbm.at[pl.ds(s * rows_per, rows_per)],
                        agg_sh.at[pl.ds(s * rows_per, rows_per)])
        if tail_n:
            @pl.when(s == 0)
            def _():
                pltpu.sync_copy(x_hbm.at[pl.ds(tail_r0, tail_n)],
                                agg_sh.at[pl.ds(tail_r0, tail_n)])
        pltpu.make_async_copy(src_hbm.at[pl.ds(w * epw, epw)], src_v,
                              sem_i).wait()
        plsc.subcore_barrier()

        blk0 = w * bpw

        def gather(b, buf, sem):
            return pltpu.make_async_copy(
                x_hbm.at[src_v.at[pl.ds(b * EBLK, EBLK)]], buf, sem)

        def dstcopy(b, buf, sem):
            return pltpu.make_async_copy(dst_hbm.at[pl.ds(blk0 + b, 1)], buf,
                                         sem)

        def scatter(buf, db):
            pltpu.sync_copy(buf, agg_sh.at[db.at[0]], add=True)

        gather(0, m_a, sem_a).start()
        dstcopy(0, db_a, sem_da).start()

        @pl.loop(0, nk2)
        def _(k2):
            b0 = 2 * k2
            b1 = b0 + 1
            gather(b1, m_b, sem_b).start()
            dstcopy(b1, db_b, sem_db).start()
            gather(b0, m_a, sem_a).wait()
            dstcopy(b0, db_a, sem_da).wait()
            scatter(m_a, db_a)

            @pl.when(k2 < nk2 - 1)
            def _():
                gather(b0 + 2, m_a, sem_a).start()
                dstcopy(b0 + 2, db_a, sem_da).start()
            gather(b1, m_b, sem_b).wait()
            dstcopy(b1, db_b, sem_db).wait()
            scatter(m_b, db_b)

        plsc.subcore_barrier()

        pltpu.sync_copy(agg_sh.at[pl.ds(s * rows_per, rows_per)],
                        out_hbm.at[c, pl.ds(s * rows_per, rows_per)])
        if tail_n:
            @pl.when(s == 0)
            def _():
                pltpu.sync_copy(agg_sh.at[pl.ds(tail_r0, tail_n)],
                                out_hbm.at[c, pl.ds(tail_r0, tail_n)])

    return sc_kernel(x, src, dst)


def _tc_body(scale_ref, x_ref, p0_ref, p1_ref, w1_ref, b1_ref, w2_ref, b2_ref,
             o_ref):
    h = x_ref[...] * scale_ref[0, 0] + p0_ref[...] + p1_ref[...]
    h = jnp.dot(h, w1_ref[...], preferred_element_type=jnp.float32,
                precision=lax.Precision.HIGHEST) + b1_ref[...]
    h = jnp.maximum(h, 0.0)
    o_ref[...] = jnp.dot(h, w2_ref[...], preferred_element_type=jnp.float32,
                         precision=lax.Precision.HIGHEST) + b2_ref[...]


def _tc_mlp(x, p0, p1, W1, b1, W2, b2, eps):
    n, d = x.shape
    blk = 1000
    grid = (n // blk,)
    scale = (eps - 1.0).reshape(1, 1)
    return pl.pallas_call(
        _tc_body,
        grid=grid,
        in_specs=[
            pl.BlockSpec((1, 1), lambda i: (0, 0)),
            pl.BlockSpec((blk, d), lambda i: (i, 0)),
            pl.BlockSpec((blk, d), lambda i: (i, 0)),
            pl.BlockSpec((blk, d), lambda i: (i, 0)),
            pl.BlockSpec((d, d), lambda i: (0, 0)),
            pl.BlockSpec((1, d), lambda i: (0, 0)),
            pl.BlockSpec((d, d), lambda i: (0, 0)),
            pl.BlockSpec((1, d), lambda i: (0, 0)),
        ],
        out_specs=pl.BlockSpec((blk, d), lambda i: (i, 0)),
        out_shape=jax.ShapeDtypeStruct((n, d), jnp.float32),
    )(scale, x, p0, p1, W1, b1.reshape(1, d), W2, b2.reshape(1, d))


def kernel(x, edge_index, W1, b1, W2, b2, eps):
    n = x.shape[0]
    e = edge_index.shape[1]
    nw = NC * NS
    unit = 2 * nw * EBLK
    e_pad = ((e + unit - 1) // unit) * unit
    pad = e_pad - e
    iot = jnp.arange(pad, dtype=jnp.int32)
    src = jnp.concatenate([edge_index[0], iot % n])
    dst = jnp.concatenate([edge_index[1], n + (iot % EBLK)])
    dst = dst.reshape(e_pad // EBLK, EBLK)
    partials = _sc_aggregate(x, src, dst, n)
    return _tc_mlp(x, partials[0], partials[1], W1, b1, W2, b2, eps)


# speedup vs baseline: 3.5076x; 1.0451x over previous
import functools

import jax
import jax.numpy as jnp
from jax import lax
from jax.experimental import pallas as pl
from jax.experimental.pallas import tpu as pltpu
from jax.experimental.pallas import tpu_sc as plsc

NC = 2
NS = 16
EBLK = 96
RING = 3


def _sc_aggregate(x, src, dst, n):
    d = x.shape[1]
    nw = NC * NS
    nblk = dst.shape[0]
    bpw = nblk // nw
    nk = bpw // RING
    epw = bpw * EBLK

    rows_per = (n // NS) & ~7
    tail_r0 = rows_per * NS
    tail_n = n - tail_r0

    mesh = plsc.VectorSubcoreMesh(core_axis_name="c", subcore_axis_name="s")

    @functools.partial(
        pl.kernel,
        mesh=mesh,
        out_type=jax.ShapeDtypeStruct((NC, n, d), jnp.float32),
        scratch_types=(
            [pltpu.VMEM((epw,), jnp.int32)]
            + [pltpu.VMEM((1, EBLK), jnp.int32) for _ in range(RING)]
            + [pltpu.VMEM((EBLK, d), jnp.float32) for _ in range(RING)]
            + [pltpu.VMEM_SHARED((n + EBLK, d), jnp.float32)]
            + [pltpu.SemaphoreType.DMA for _ in range(3 * RING + 1)]
        ),
    )
    def sc_kernel(x_hbm, src_hbm, dst_hbm, out_hbm, src_v,
                  db0, db1, db2, m0, m1, m2, agg_sh,
                  g0, g1, g2, d0, d1, d2, s0, s1, s2, sem_i):
        dbs = (db0, db1, db2)
        ms = (m0, m1, m2)
        gsem = (g0, g1, g2)
        dsem = (d0, d1, d2)
        ssem = (s0, s1, s2)
        c = lax.axis_index("c")
        s = lax.axis_index("s")
        w = c * NS + s

        pltpu.make_async_copy(src_hbm.at[pl.ds(w * epw, epw)], src_v,
                              sem_i).start()

        pltpu.sync_copy(x_hbm.at[pl.ds(s * rows_per, rows_per)],
                        agg_sh.at[pl.ds(s * rows_per, rows_per)])
        if tail_n:
            @pl.when(s == 0)
            def _():
                pltpu.sync_copy(x_hbm.at[pl.ds(tail_r0, tail_n)],
                                agg_sh.at[pl.ds(tail_r0, tail_n)])
        pltpu.make_async_copy(src_hbm.at[pl.ds(w * epw, epw)], src_v,
                              sem_i).wait()
        plsc.subcore_barrier()

        blk0 = w * bpw

        def gather(t, i):
            return pltpu.make_async_copy(
                x_hbm.at[src_v.at[pl.ds(t * EBLK, EBLK)]], ms[i], gsem[i])

        def dstcopy(t, i):
            return pltpu.make_async_copy(dst_hbm.at[pl.ds(blk0 + t, 1)],
                                         dbs[i], dsem[i])

        def scat_start(i):
            pltpu.async_copy(ms[i], agg_sh.at[dbs[i].at[0]], ssem[i],
                             add=True)

        def scat_wait(i):
            pltpu.make_async_copy(ms[i], agg_sh.at[dbs[i].at[0]],
                                  ssem[i]).wait()

        def start_pipe(t, i):
            dstcopy(t, i).start()
            gather(t, i).start()

        start_pipe(0, 0)

        @pl.loop(0, nk)
        def _(k):
            for i in range(RING):
                t = RING * k + i
                i1 = (i + 1) % RING
                tn = t + 1

                @pl.when(tn < bpw)
                def _():
                    @pl.when(t >= 2)
                    def _():
                        scat_wait(i1)
                    start_pipe(tn, i1)
                gather(t, i).wait()
                dstcopy(t, i).wait()
                scat_start(i)

        for i in range(RING):
            scat_wait(i)
        plsc.subcore_barrier()

        pltpu.sync_copy(agg_sh.at[pl.ds(s * rows_per, rows_per)],
                        out_hbm.at[c, pl.ds(s * rows_per, rows_per)])
        if tail_n:
            @pl.when(s == 0)
            def _():
                pltpu.sync_copy(agg_sh.at[pl.ds(tail_r0, tail_n)],
                                out_hbm.at[c, pl.ds(tail_r0, tail_n)])

    return sc_kernel(x, src, dst)


def _tc_body(scale_ref, x_ref, p0_ref, p1_ref, w1_ref, b1_ref, w2_ref, b2_ref,
             o_ref):
    h = x_ref[...] * scale_ref[0, 0] + p0_ref[...] + p1_ref[...]
    h = jnp.dot(h, w1_ref[...], preferred_element_type=jnp.float32,
                precision=lax.Precision.HIGHEST) + b1_ref[...]
    h = jnp.maximum(h, 0.0)
    o_ref[...] = jnp.dot(h, w2_ref[...], preferred_element_type=jnp.float32,
                         precision=lax.Precision.HIGHEST) + b2_ref[...]


def _tc_mlp(x, p0, p1, W1, b1, W2, b2, eps):
    n, d = x.shape
    blk = 1000
    grid = (n // blk,)
    scale = (eps - 1.0).reshape(1, 1)
    return pl.pallas_call(
        _tc_body,
        grid=grid,
        in_specs=[
            pl.BlockSpec((1, 1), lambda i: (0, 0)),
            pl.BlockSpec((blk, d), lambda i: (i, 0)),
            pl.BlockSpec((blk, d), lambda i: (i, 0)),
            pl.BlockSpec((blk, d), lambda i: (i, 0)),
            pl.BlockSpec((d, d), lambda i: (0, 0)),
            pl.BlockSpec((1, d), lambda i: (0, 0)),
            pl.BlockSpec((d, d), lambda i: (0, 0)),
            pl.BlockSpec((1, d), lambda i: (0, 0)),
        ],
        out_specs=pl.BlockSpec((blk, d), lambda i: (i, 0)),
        out_shape=jax.ShapeDtypeStruct((n, d), jnp.float32),
    )(scale, x, p0, p1, W1, b1.reshape(1, d), W2, b2.reshape(1, d))


def kernel(x, edge_index, W1, b1, W2, b2, eps):
    n = x.shape[0]
    e = edge_index.shape[1]
    nw = NC * NS
    unit = RING * nw * EBLK
    e_pad = ((e + unit - 1) // unit) * unit
    pad = e_pad - e
    iot = jnp.arange(pad, dtype=jnp.int32)
    src = jnp.concatenate([edge_index[0], iot % n])
    dst = jnp.concatenate([edge_index[1], n + (iot % EBLK)])
    dst = dst.reshape(e_pad // EBLK, EBLK)
    partials = _sc_aggregate(x, src, dst, n)
    return _tc_mlp(x, partials[0], partials[1], W1, b1, W2, b2, eps)


# speedup vs baseline: 4.1972x; 1.1966x over previous
import functools

import jax
import jax.numpy as jnp
from jax import lax
from jax.experimental import pallas as pl
from jax.experimental.pallas import tpu as pltpu
from jax.experimental.pallas import tpu_sc as plsc

NC = 2
NS = 16
EBLK = 96
RING = 3


def _sc_aggregate(x, src, dst, n):
    d = x.shape[1]
    nw = NC * NS
    nblk = dst.shape[0]
    bpw = nblk // nw
    nk = bpw // RING
    epw = bpw * EBLK

    rows_per = (n // NS) & ~7
    tail_r0 = rows_per * NS
    tail_n = n - tail_r0

    mesh = plsc.VectorSubcoreMesh(core_axis_name="c", subcore_axis_name="s")

    @functools.partial(
        pl.kernel,
        mesh=mesh,
        out_type=jax.ShapeDtypeStruct((NC, n, d), jnp.float32),
        scratch_types=(
            [pltpu.VMEM((epw,), jnp.int32)]
            + [pltpu.VMEM((1, EBLK), jnp.int32) for _ in range(RING)]
            + [pltpu.VMEM((EBLK, d), jnp.float32) for _ in range(RING)]
            + [pltpu.VMEM_SHARED((n + EBLK, d), jnp.float32)]
            + [pltpu.SemaphoreType.DMA for _ in range(3 * RING + 1)]
        ),
    )
    def sc_kernel(x_hbm, src_hbm, dst_hbm, out_hbm, src_v,
                  db0, db1, db2, m0, m1, m2, agg_sh,
                  g0, g1, g2, d0, d1, d2, s0, s1, s2, sem_i):
        dbs = (db0, db1, db2)
        ms = (m0, m1, m2)
        gsem = (g0, g1, g2)
        dsem = (d0, d1, d2)
        ssem = (s0, s1, s2)
        c = lax.axis_index("c")
        s = lax.axis_index("s")
        w = c * NS + s

        pltpu.make_async_copy(src_hbm.at[pl.ds(w * epw, epw)], src_v,
                              sem_i).start()

        pltpu.sync_copy(x_hbm.at[pl.ds(s * rows_per, rows_per)],
                        agg_sh.at[pl.ds(s * rows_per, rows_per)])
        if tail_n:
            @pl.when(s == 0)
            def _():
                pltpu.sync_copy(x_hbm.at[pl.ds(tail_r0, tail_n)],
                                agg_sh.at[pl.ds(tail_r0, tail_n)])
        pltpu.make_async_copy(src_hbm.at[pl.ds(w * epw, epw)], src_v,
                              sem_i).wait()
        plsc.subcore_barrier()

        blk0 = w * bpw

        def gather(t, i):
            return pltpu.make_async_copy(
                x_hbm.at[src_v.at[pl.ds(t * EBLK, EBLK)]], ms[i], gsem[i])

        def dstcopy(t, i):
            return pltpu.make_async_copy(dst_hbm.at[pl.ds(blk0 + t, 1)],
                                         dbs[i], dsem[i])

        def scat_start(i):
            pltpu.async_copy(ms[i], agg_sh.at[dbs[i].at[0]], ssem[i],
                             add=True)

        def scat_wait(i):
            pltpu.make_async_copy(ms[i], agg_sh.at[dbs[i].at[0]],
                                  ssem[i]).wait()

        def start_pipe(t, i):
            dstcopy(t, i).start()
            gather(t, i).start()

        start_pipe(0, 0)

        @pl.loop(0, nk)
        def _(k):
            for i in range(RING):
                t = RING * k + i
                i1 = (i + 1) % RING
                tn = t + 1

                @pl.when(tn < bpw)
                def _():
                    @pl.when(t >= 2)
                    def _():
                        scat_wait(i1)
                    start_pipe(tn, i1)
                gather(t, i).wait()
                dstcopy(t, i).wait()
                scat_start(i)

        for i in range(RING):
            scat_wait(i)
        plsc.subcore_barrier()

        pltpu.sync_copy(agg_sh.at[pl.ds(s * rows_per, rows_per)],
                        out_hbm.at[c, pl.ds(s * rows_per, rows_per)])
        if tail_n:
            @pl.when(s == 0)
            def _():
                pltpu.sync_copy(agg_sh.at[pl.ds(tail_r0, tail_n)],
                                out_hbm.at[c, pl.ds(tail_r0, tail_n)])

    return sc_kernel(x, src, dst)


def _tc_body(scale_ref, x_ref, p_ref, w1_ref, b1_ref, w2_ref, b2_ref,
             o_ref):
    h = x_ref[...] * scale_ref[0, 0] + p_ref[0] + p_ref[1]
    h = jnp.dot(h, w1_ref[...],
                preferred_element_type=jnp.float32) + b1_ref[...]
    h = jnp.maximum(h, 0.0)
    o_ref[...] = jnp.dot(h, w2_ref[...],
                         preferred_element_type=jnp.float32) + b2_ref[...]


def _tc_mlp(x, p, W1, b1, W2, b2, eps):
    n, d = x.shape
    blk = 1000
    grid = (n // blk,)
    scale = (eps - 1.0).reshape(1, 1)
    return pl.pallas_call(
        _tc_body,
        grid=grid,
        in_specs=[
            pl.BlockSpec((1, 1), lambda i: (0, 0)),
            pl.BlockSpec((blk, d), lambda i: (i, 0)),
            pl.BlockSpec((2, blk, d), lambda i: (0, i, 0)),
            pl.BlockSpec((d, d), lambda i: (0, 0)),
            pl.BlockSpec((1, d), lambda i: (0, 0)),
            pl.BlockSpec((d, d), lambda i: (0, 0)),
            pl.BlockSpec((1, d), lambda i: (0, 0)),
        ],
        out_specs=pl.BlockSpec((blk, d), lambda i: (i, 0)),
        out_shape=jax.ShapeDtypeStruct((n, d), jnp.float32),
    )(scale, x, p, W1, b1.reshape(1, d), W2, b2.reshape(1, d))


def kernel(x, edge_index, W1, b1, W2, b2, eps):
    n = x.shape[0]
    e = edge_index.shape[1]
    nw = NC * NS
    unit = RING * nw * EBLK
    e_pad = ((e + unit - 1) // unit) * unit
    pad = e_pad - e
    iot = jnp.arange(pad, dtype=jnp.int32)
    src = jnp.concatenate([edge_index[0], iot % n])
    dst = jnp.concatenate([edge_index[1], n + (iot % EBLK)])
    dst = dst.reshape(e_pad // EBLK, EBLK)
    partials = _sc_aggregate(x, src, dst, n)
    return _tc_mlp(x, partials, W1, b1, W2, b2, eps)


# speedup vs baseline: 4.3426x; 1.0346x over previous
import functools

import jax
import jax.numpy as jnp
import numpy as np
from jax import lax
from jax.experimental import pallas as pl
from jax.experimental.pallas import tpu as pltpu
from jax.experimental.pallas import tpu_sc as plsc

NC = 2
NS = 16
EBLK = 64
RING = 3


def _sc_aggregate(x, ei, pad_src, pad_dst, n, e):
    d = x.shape[1]
    nw = NC * NS
    pad = pad_src.shape[0]
    nblk = (e + pad) // EBLK
    bpw = nblk // nw
    nk = bpw // RING
    epw = bpw * EBLK
    nreal = e // EBLK
    wm = nreal // bpw
    real_in_wm = e - wm * epw

    rows_per = (n // NS) & ~7
    tail_r0 = rows_per * NS
    tail_n = n - tail_r0

    mesh = plsc.VectorSubcoreMesh(core_axis_name="c", subcore_axis_name="s")

    @functools.partial(
        pl.kernel,
        mesh=mesh,
        out_type=jax.ShapeDtypeStruct((NC, n, d), jnp.float32),
        scratch_types=(
            [pltpu.VMEM((2, epw), jnp.int32)]
            + [pltpu.VMEM((EBLK,), jnp.int32) for _ in range(RING)]
            + [pltpu.VMEM((EBLK, d), jnp.float32) for _ in range(RING)]
            + [pltpu.VMEM_SHARED((n + EBLK, d), jnp.float32)]
            + [pltpu.SemaphoreType.DMA for _ in range(2 * RING + 1)]
        ),
    )
    def sc_kernel(x_hbm, ei_hbm, ps_hbm, pd_hbm, out_hbm, sd_v,
                  db0, db1, db2, m0, m1, m2, agg_sh,
                  g0, g1, g2, s0, s1, s2, sem_i):
        dbs = (db0, db1, db2)
        ms = (m0, m1, m2)
        gsem = (g0, g1, g2)
        ssem = (s0, s1, s2)
        c = lax.axis_index("c")
        s = lax.axis_index("s")
        w = c * NS + s

        @pl.when(w < wm)
        def _():
            pltpu.make_async_copy(ei_hbm.at[:, pl.ds(w * epw, epw)], sd_v,
                                  sem_i).start()
        @pl.when(w == wm)
        def _():
            if real_in_wm:
                pltpu.make_async_copy(
                    ei_hbm.at[:, pl.ds(wm * epw, real_in_wm)],
                    sd_v.at[:, pl.ds(0, real_in_wm)], sem_i).start()
            if epw > real_in_wm:
                pltpu.make_async_copy(
                    ps_hbm.at[pl.ds(0, epw - real_in_wm)],
                    sd_v.at[0, pl.ds(real_in_wm, epw - real_in_wm)],
                    sem_i).start()
                pltpu.make_async_copy(
                    pd_hbm.at[pl.ds(0, epw - real_in_wm)],
                    sd_v.at[1, pl.ds(real_in_wm, epw - real_in_wm)],
                    sem_i).start()
        @pl.when(w > wm)
        def _():
            pltpu.make_async_copy(ps_hbm.at[pl.ds(w * epw - e, epw)],
                                  sd_v.at[0], sem_i).start()
            pltpu.make_async_copy(pd_hbm.at[pl.ds(w * epw - e, epw)],
                                  sd_v.at[1], sem_i).start()

        pltpu.sync_copy(x_hbm.at[pl.ds(s * rows_per, rows_per)],
                        agg_sh.at[pl.ds(s * rows_per, rows_per)])
        if tail_n:
            @pl.when(s == 0)
            def _():
                pltpu.sync_copy(x_hbm.at[pl.ds(tail_r0, tail_n)],
                                agg_sh.at[pl.ds(tail_r0, tail_n)])
        pltpu.make_async_copy(ei_hbm.at[:, pl.ds(0, epw)], sd_v,
                              sem_i).wait()
        plsc.subcore_barrier()

        def gather(t, i):
            return pltpu.make_async_copy(
                x_hbm.at[sd_v.at[0, pl.ds(t * EBLK, EBLK)]], ms[i], gsem[i])

        def fill_db(t, i):
            for j in range(EBLK // 16):
                dbs[i][pl.ds(16 * j, 16)] = (
                    sd_v[1, pl.ds(t * EBLK + 16 * j, 16)])

        def scat_start(i):
            pltpu.async_copy(ms[i], agg_sh.at[dbs[i]], ssem[i], add=True)

        def scat_wait(i):
            pltpu.make_async_copy(ms[i], agg_sh.at[dbs[i]], ssem[i]).wait()

        def start_pipe(t, i):
            gather(t, i).start()
            fill_db(t, i)

        start_pipe(0, 0)

        @pl.loop(0, nk)
        def _(k):
            for i in range(RING):
                t = RING * k + i
                i1 = (i + 1) % RING
                tn = t + 1

                @pl.when(tn < bpw)
                def _():
                    @pl.when(t >= 2)
                    def _():
                        scat_wait(i1)
                    start_pipe(tn, i1)
                gather(t, i).wait()
                scat_start(i)

        for i in range(RING):
            scat_wait(i)
        plsc.subcore_barrier()

        pltpu.sync_copy(agg_sh.at[pl.ds(s * rows_per, rows_per)],
                        out_hbm.at[c, pl.ds(s * rows_per, rows_per)])
        if tail_n:
            @pl.when(s == 0)
            def _():
                pltpu.sync_copy(agg_sh.at[pl.ds(tail_r0, tail_n)],
                                out_hbm.at[c, pl.ds(tail_r0, tail_n)])

    return sc_kernel(x, ei, pad_src, pad_dst)


def _tc_body(scale_ref, x_ref, p_ref, w1_ref, b1_ref, w2_ref, b2_ref,
             o_ref):
    h = x_ref[...] * scale_ref[0, 0] + p_ref[0] + p_ref[1]
    h = jnp.dot(h, w1_ref[...],
                preferred_element_type=jnp.float32) + b1_ref[...]
    h = jnp.maximum(h, 0.0)
    o_ref[...] = jnp.dot(h, w2_ref[...],
                         preferred_element_type=jnp.float32) + b2_ref[...]


def _tc_mlp(x, p, W1, b1, W2, b2, eps):
    n, d = x.shape
    blk = 1000
    grid = (n // blk,)
    scale = (eps - 1.0).reshape(1, 1)
    return pl.pallas_call(
        _tc_body,
        grid=grid,
        in_specs=[
            pl.BlockSpec((1, 1), lambda i: (0, 0)),
            pl.BlockSpec((blk, d), lambda i: (i, 0)),
            pl.BlockSpec((2, blk, d), lambda i: (0, i, 0)),
            pl.BlockSpec((d, d), lambda i: (0, 0)),
            pl.BlockSpec((1, d), lambda i: (0, 0)),
            pl.BlockSpec((d, d), lambda i: (0, 0)),
            pl.BlockSpec((1, d), lambda i: (0, 0)),
        ],
        out_specs=pl.BlockSpec((blk, d), lambda i: (i, 0)),
        out_shape=jax.ShapeDtypeStruct((n, d), jnp.float32),
    )(scale, x, p, W1, b1.reshape(1, d), W2, b2.reshape(1, d))


def kernel(x, edge_index, W1, b1, W2, b2, eps):
    n = x.shape[0]
    e = edge_index.shape[1]
    nw = NC * NS
    unit = np.lcm(RING * nw * EBLK, 128 * nw)
    e_pad = int(-(-e // unit) * unit)
    pad = e_pad - e
    iot = np.arange(pad, dtype=np.int32)
    pad_src = jnp.asarray(iot % n)
    pad_dst = jnp.asarray(n + (iot % EBLK))
    partials = _sc_aggregate(x, edge_index, pad_src, pad_dst, n, e)
    return _tc_mlp(x, partials, W1, b1, W2, b2, eps)


# speedup vs baseline: 4.4177x; 1.0173x over previous
import functools

import jax
import jax.numpy as jnp
import numpy as np
from jax import lax
from jax.experimental import pallas as pl
from jax.experimental.pallas import tpu as pltpu
from jax.experimental.pallas import tpu_sc as plsc

NC = 2
NS = 16
EBLK = 64
RING = 3


def _sc_aggregate(x, ei, pad_src, pad_dst, n, e):
    d = x.shape[1]
    nw = NC * NS
    pad = pad_src.shape[0]
    nblk = (e + pad) // EBLK
    bpw = nblk // nw
    nk = bpw // RING
    epw = bpw * EBLK
    nreal = e // EBLK
    wm = nreal // bpw
    real_in_wm = e - wm * epw

    rows_per = (n // NS) & ~7
    tail_r0 = rows_per * NS
    tail_n = n - tail_r0

    mesh = plsc.VectorSubcoreMesh(core_axis_name="c", subcore_axis_name="s")

    @functools.partial(
        pl.kernel,
        mesh=mesh,
        out_type=jax.ShapeDtypeStruct((NC, n, d), jnp.float32),
        scratch_types=(
            [pltpu.VMEM((2, epw), jnp.int32)]
            + [pltpu.VMEM((EBLK, d), jnp.float32) for _ in range(RING)]
            + [pltpu.VMEM_SHARED((n + EBLK, d), jnp.float32)]
            + [pltpu.SemaphoreType.DMA for _ in range(2 * RING + 1)]
        ),
    )
    def sc_kernel(x_hbm, ei_hbm, ps_hbm, pd_hbm, out_hbm, sd_v,
                  m0, m1, m2, agg_sh,
                  g0, g1, g2, s0, s1, s2, sem_i):
        ms = (m0, m1, m2)
        gsem = (g0, g1, g2)
        ssem = (s0, s1, s2)
        c = lax.axis_index("c")
        s = lax.axis_index("s")
        w = c * NS + s

        @pl.when(w < wm)
        def _():
            pltpu.make_async_copy(ei_hbm.at[:, pl.ds(w * epw, epw)], sd_v,
                                  sem_i).start()
        @pl.when(w == wm)
        def _():
            if real_in_wm:
                pltpu.make_async_copy(
                    ei_hbm.at[:, pl.ds(wm * epw, real_in_wm)],
                    sd_v.at[:, pl.ds(0, real_in_wm)], sem_i).start()
            if epw > real_in_wm:
                pltpu.make_async_copy(
                    ps_hbm.at[pl.ds(0, epw - real_in_wm)],
                    sd_v.at[0, pl.ds(real_in_wm, epw - real_in_wm)],
                    sem_i).start()
                pltpu.make_async_copy(
                    pd_hbm.at[pl.ds(0, epw - real_in_wm)],
                    sd_v.at[1, pl.ds(real_in_wm, epw - real_in_wm)],
                    sem_i).start()
        @pl.when(w > wm)
        def _():
            pltpu.make_async_copy(ps_hbm.at[pl.ds(w * epw - e, epw)],
                                  sd_v.at[0], sem_i).start()
            pltpu.make_async_copy(pd_hbm.at[pl.ds(w * epw - e, epw)],
                                  sd_v.at[1], sem_i).start()

        pltpu.sync_copy(x_hbm.at[pl.ds(s * rows_per, rows_per)],
                        agg_sh.at[pl.ds(s * rows_per, rows_per)])
        if tail_n:
            @pl.when(s == 0)
            def _():
                pltpu.sync_copy(x_hbm.at[pl.ds(tail_r0, tail_n)],
                                agg_sh.at[pl.ds(tail_r0, tail_n)])
        pltpu.make_async_copy(ei_hbm.at[:, pl.ds(0, epw)], sd_v,
                              sem_i).wait()
        plsc.subcore_barrier()

        def gather(t, i):
            return pltpu.make_async_copy(
                x_hbm.at[sd_v.at[0, pl.ds(t * EBLK, EBLK)]], ms[i], gsem[i])

        def dstrow(t):
            return sd_v.at[1, pl.ds(t * EBLK, EBLK)]

        def scat_start(t, i):
            pltpu.async_copy(ms[i], agg_sh.at[dstrow(t)], ssem[i], add=True)

        def scat_wait(t, i):
            pltpu.make_async_copy(ms[i], agg_sh.at[dstrow(t)],
                                  ssem[i]).wait()

        gather(0, 0).start()

        @pl.loop(0, nk)
        def _(k):
            for i in range(RING):
                t = RING * k + i
                i1 = (i + 1) % RING
                tn = t + 1

                @pl.when(tn < bpw)
                def _():
                    @pl.when(t >= 2)
                    def _():
                        scat_wait(t - 2, i1)
                    gather(tn, i1).start()
                gather(t, i).wait()
                scat_start(t, i)

        for i in range(RING):
            t_last = bpw - RING + (i - bpw) % RING
            scat_wait(t_last, i)
        plsc.subcore_barrier()

        pltpu.sync_copy(agg_sh.at[pl.ds(s * rows_per, rows_per)],
                        out_hbm.at[c, pl.ds(s * rows_per, rows_per)])
        if tail_n:
            @pl.when(s == 0)
            def _():
                pltpu.sync_copy(agg_sh.at[pl.ds(tail_r0, tail_n)],
                                out_hbm.at[c, pl.ds(tail_r0, tail_n)])

    return sc_kernel(x, ei, pad_src, pad_dst)


def _tc_body(scale_ref, x_ref, p_ref, w1_ref, b1_ref, w2_ref, b2_ref,
             o_ref):
    h = x_ref[...] * scale_ref[0, 0] + p_ref[0] + p_ref[1]
    h = jnp.dot(h, w1_ref[...],
                preferred_element_type=jnp.float32) + b1_ref[...]
    h = jnp.maximum(h, 0.0)
    o_ref[...] = jnp.dot(h, w2_ref[...],
                         preferred_element_type=jnp.float32) + b2_ref[...]


def _tc_mlp(x, p, W1, b1, W2, b2, eps):
    n, d = x.shape
    blk = 2000
    grid = (n // blk,)
    scale = (eps - 1.0).reshape(1, 1)
    return pl.pallas_call(
        _tc_body,
        grid=grid,
        in_specs=[
            pl.BlockSpec((1, 1), lambda i: (0, 0)),
            pl.BlockSpec((blk, d), lambda i: (i, 0)),
            pl.BlockSpec((2, blk, d), lambda i: (0, i, 0)),
            pl.BlockSpec((d, d), lambda i: (0, 0)),
            pl.BlockSpec((1, d), lambda i: (0, 0)),
            pl.BlockSpec((d, d), lambda i: (0, 0)),
            pl.BlockSpec((1, d), lambda i: (0, 0)),
        ],
        out_specs=pl.BlockSpec((blk, d), lambda i: (i, 0)),
        out_shape=jax.ShapeDtypeStruct((n, d), jnp.float32),
    )(scale, x, p, W1, b1.reshape(1, d), W2, b2.reshape(1, d))


def kernel(x, edge_index, W1, b1, W2, b2, eps):
    n = x.shape[0]
    e = edge_index.shape[1]
    nw = NC * NS
    unit = np.lcm(RING * nw * EBLK, 128 * nw)
    e_pad = int(-(-e // unit) * unit)
    pad = e_pad - e
    iot = np.arange(pad, dtype=np.int32)
    pad_src = jnp.asarray(iot % n)
    pad_dst = jnp.asarray(n + (iot % EBLK))
    partials = _sc_aggregate(x, edge_index, pad_src, pad_dst, n, e)
    return _tc_mlp(x, partials, W1, b1, W2, b2, eps)


# speedup vs baseline: 4.6591x; 1.0547x over previous
import functools

import jax
import jax.numpy as jnp
import numpy as np
from jax import lax
from jax.experimental import pallas as pl
from jax.experimental.pallas import tpu as pltpu
from jax.experimental.pallas import tpu_sc as plsc

NC = 2
NS = 16
EBLK = 64
RING = 3


def _sc_aggregate(x, ei, pad_src, pad_dst, n, e):
    d = x.shape[1]
    nw = NC * NS
    pad = pad_src.shape[0]
    nblk = (e + pad) // EBLK
    bpw = nblk // nw
    nk = bpw // RING
    epw = bpw * EBLK
    nreal = e // EBLK
    wm = nreal // bpw
    real_in_wm = e - wm * epw

    rows_per = (n // NS) & ~7
    tail_r0 = rows_per * NS
    tail_n = n - tail_r0

    mesh = plsc.VectorSubcoreMesh(core_axis_name="c", subcore_axis_name="s")

    @functools.partial(
        pl.kernel,
        mesh=mesh,
        out_type=jax.ShapeDtypeStruct((NC, n, d), jnp.float32),
        scratch_types=(
            [pltpu.VMEM((2, epw), jnp.int32)]
            + [pltpu.VMEM((EBLK, d), jnp.float32) for _ in range(RING)]
            + [pltpu.VMEM_SHARED((n + EBLK, d), jnp.float32)]
            + [pltpu.SemaphoreType.DMA for _ in range(2 * RING + 1)]
        ),
    )
    def sc_kernel(x_hbm, ei_hbm, ps_hbm, pd_hbm, out_hbm, sd_v,
                  m0, m1, m2, agg_sh,
                  g0, g1, g2, s0, s1, s2, sem_i):
        ms = (m0, m1, m2)
        gsem = (g0, g1, g2)
        ssem = (s0, s1, s2)
        c = lax.axis_index("c")
        s = lax.axis_index("s")
        w = c * NS + s

        @pl.when(w < wm)
        def _():
            pltpu.make_async_copy(ei_hbm.at[:, pl.ds(w * epw, epw)], sd_v,
                                  sem_i).start()
        @pl.when(w == wm)
        def _():
            if real_in_wm:
                pltpu.make_async_copy(
                    ei_hbm.at[:, pl.ds(wm * epw, real_in_wm)],
                    sd_v.at[:, pl.ds(0, real_in_wm)], sem_i).start()
            if epw > real_in_wm:
                pltpu.make_async_copy(
                    ps_hbm.at[pl.ds(0, epw - real_in_wm)],
                    sd_v.at[0, pl.ds(real_in_wm, epw - real_in_wm)],
                    sem_i).start()
                pltpu.make_async_copy(
                    pd_hbm.at[pl.ds(0, epw - real_in_wm)],
                    sd_v.at[1, pl.ds(real_in_wm, epw - real_in_wm)],
                    sem_i).start()
        @pl.when(w > wm)
        def _():
            pltpu.make_async_copy(ps_hbm.at[pl.ds(w * epw - e, epw)],
                                  sd_v.at[0], sem_i).start()
            pltpu.make_async_copy(pd_hbm.at[pl.ds(w * epw - e, epw)],
                                  sd_v.at[1], sem_i).start()

        pltpu.sync_copy(x_hbm.at[pl.ds(s * rows_per, rows_per)],
                        agg_sh.at[pl.ds(s * rows_per, rows_per)])
        if tail_n:
            @pl.when(s == 0)
            def _():
                pltpu.sync_copy(x_hbm.at[pl.ds(tail_r0, tail_n)],
                                agg_sh.at[pl.ds(tail_r0, tail_n)])
        pltpu.make_async_copy(ei_hbm.at[:, pl.ds(0, epw)], sd_v,
                              sem_i).wait()
        plsc.subcore_barrier()

        def gather(t, i):
            return pltpu.make_async_copy(
                x_hbm.at[sd_v.at[0, pl.ds(t * EBLK, EBLK)]], ms[i], gsem[i])

        def dstrow(t):
            return sd_v.at[1, pl.ds(t * EBLK, EBLK)]

        def scat_start(t, i):
            pltpu.async_copy(ms[i], agg_sh.at[dstrow(t)], ssem[i], add=True)

        def scat_wait(t, i):
            pltpu.make_async_copy(ms[i], agg_sh.at[dstrow(t)],
                                  ssem[i]).wait()

        gather(0, 0).start()
        gather(1, 1).start()

        @pl.loop(0, nk)
        def _(k):
            for i in range(RING):
                t = RING * k + i
                i2 = (i + 2) % RING
                tn = t + 2

                @pl.when(tn < bpw)
                def _():
                    @pl.when(t >= 1)
                    def _():
                        scat_wait(t - 1, i2)
                    gather(tn, i2).start()
                gather(t, i).wait()
                scat_start(t, i)

        for i in range(RING):
            t_last = bpw - RING + (i - bpw) % RING
            scat_wait(t_last, i)
        plsc.subcore_barrier()

        pltpu.sync_copy(agg_sh.at[pl.ds(s * rows_per, rows_per)],
                        out_hbm.at[c, pl.ds(s * rows_per, rows_per)])
        if tail_n:
            @pl.when(s == 0)
            def _():
                pltpu.sync_copy(agg_sh.at[pl.ds(tail_r0, tail_n)],
                                out_hbm.at[c, pl.ds(tail_r0, tail_n)])

    return sc_kernel(x, ei, pad_src, pad_dst)


def _tc_body(scale_ref, x_ref, p_ref, w1_ref, b1_ref, w2_ref, b2_ref,
             o_ref):
    h = x_ref[...] * scale_ref[0, 0] + p_ref[0] + p_ref[1]
    h = jnp.dot(h, w1_ref[...],
                preferred_element_type=jnp.float32) + b1_ref[...]
    h = jnp.maximum(h, 0.0)
    o_ref[...] = jnp.dot(h, w2_ref[...],
                         preferred_element_type=jnp.float32) + b2_ref[...]


def _tc_mlp(x, p, W1, b1, W2, b2, eps):
    n, d = x.shape
    blk = 2000
    grid = (n // blk,)
    scale = (eps - 1.0).reshape(1, 1)
    return pl.pallas_call(
        _tc_body,
        grid=grid,
        in_specs=[
            pl.BlockSpec((1, 1), lambda i: (0, 0)),
            pl.BlockSpec((blk, d), lambda i: (i, 0)),
            pl.BlockSpec((2, blk, d), lambda i: (0, i, 0)),
            pl.BlockSpec((d, d), lambda i: (0, 0)),
            pl.BlockSpec((1, d), lambda i: (0, 0)),
            pl.BlockSpec((d, d), lambda i: (0, 0)),
            pl.BlockSpec((1, d), lambda i: (0, 0)),
        ],
        out_specs=pl.BlockSpec((blk, d), lambda i: (i, 0)),
        out_shape=jax.ShapeDtypeStruct((n, d), jnp.float32),
    )(scale, x, p, W1, b1.reshape(1, d), W2, b2.reshape(1, d))


def kernel(x, edge_index, W1, b1, W2, b2, eps):
    n = x.shape[0]
    e = edge_index.shape[1]
    nw = NC * NS
    unit = np.lcm(RING * nw * EBLK, 128 * nw)
    e_pad = int(-(-e // unit) * unit)
    pad = e_pad - e
    iot = np.arange(pad, dtype=np.int32)
    pad_src = jnp.asarray(iot % n)
    pad_dst = jnp.asarray(n + (iot % EBLK))
    partials = _sc_aggregate(x, edge_index, pad_src, pad_dst, n, e)
    return _tc_mlp(x, partials, W1, b1, W2, b2, eps)
